# baseline (speedup 1.0000x reference)
import functools

import jax
import jax.numpy as jnp
from jax import lax
from jax.experimental import pallas as pl
from jax.experimental.pallas import tpu as pltpu
from jax.experimental.pallas import tpu_sc as plsc

_INFO = plsc.get_sparse_core_info()
_NC, _NS = _INFO.num_cores, _INFO.num_subcores
_NW = _NC * _NS
_L = 16

_CHUNK = 128
_NBUF = 4


@functools.partial(jax.jit, static_argnames=("n", "d", "out_shape"))
def _emb_lookup(tp, idx1d, dummy, *, n, d, out_shape):
    per_w = n // _NW
    n_chunks = per_w // _CHUNK
    B, T, Hp, Wp, _ = out_shape
    hh = Hp // 2
    assert hh * Wp == _CHUNK and n_chunks * _NW == 2 * B * T
    assert (n_chunks - 2 * _NBUF) % _NBUF == 0 and n_chunks > 3 * _NBUF
    mesh = plsc.VectorSubcoreMesh(core_axis_name="c", subcore_axis_name="s")

    @functools.partial(
        pl.kernel,
        mesh=mesh,
        out_type=jax.ShapeDtypeStruct(out_shape, jnp.float32),
        scratch_types=[
            pltpu.VMEM((per_w,), jnp.int32),
            pltpu.VMEM((_NBUF, _CHUNK // 2), jnp.int32),
            pltpu.VMEM((_NBUF, _CHUNK // 2, 2 * d), jnp.float32),
            pltpu.VMEM((_NBUF, hh, Wp, d), jnp.float32),
            pltpu.SemaphoreType.DMA,
            pltpu.SemaphoreType.DMA,
            pltpu.SemaphoreType.DMA,
            pltpu.SemaphoreType.DMA,
            pltpu.SemaphoreType.DMA,
            pltpu.SemaphoreType.DMA,
            pltpu.SemaphoreType.DMA,
            pltpu.SemaphoreType.DMA,
        ],
    )
    def k(tp_hbm, idx_hbm, dummy_hbm, out5d, idx_all, pid_v, pairs_f,
          stage_v, *sems):
        sem_g = sems[:_NBUF]
        sem_w = sems[_NBUF:]
        wid = lax.axis_index("s") * _NC + lax.axis_index("c")
        idx0 = wid * per_w
        chunk0 = wid * n_chunks
        lane = lax.iota(jnp.int32, _L)
        low_half = lane < 8
        ev_sel = jnp.arange(0, 2 * _L, 2, dtype=jnp.int32) % _L
        od_sel = ev_sel + 1

        def deinterleave(vv, sel):
            return vv.at[sel].get(mode="promise_in_bounds")

        def fire_gather(j, b):
            base = j * _CHUNK
            for g in range(_CHUNK // (2 * _L)):
                v0 = idx_all[pl.ds(base + 2 * _L * g, _L)]
                v1 = idx_all[pl.ds(base + 2 * _L * g + _L, _L)]
                ev = jnp.where(low_half, deinterleave(v0, ev_sel),
                               deinterleave(v1, ev_sel))
                od = jnp.where(low_half, deinterleave(v0, od_sel),
                               deinterleave(v1, od_sel))
                pid_v[b, pl.ds(g * _L, _L)] = ev * 8 + od
            pltpu.make_async_copy(
                tp_hbm.at[pid_v.at[b]], pairs_f.at[b], sem_g[b]).start()

        def wait_gather(b):
            pltpu.make_async_copy(dummy_hbm, pairs_f.at[b], sem_g[b]).wait()

        def repack(b):
            def move(r_, carry):
                for u in range(2):
                    pr = 2 * r_ + u
                    for half in range(2):
                        row = 2 * pr + half
                        rh = row // Wp
                        rw = lax.rem(row, Wp)
                        for c in range(0, d, _L):
                            stage_v[b, rh, rw, pl.ds(c, _L)] = \
                                pairs_f[b, pr, pl.ds(half * d + c, _L)]
                return carry

            lax.fori_loop(0, _CHUNK // 4, move, 0)

        def fire_write(j, b):
            g = chunk0 + j
            bi = g // (2 * T)
            r = g % (2 * T)
            ti = r // 2
            hi = (r % 2) * hh
            pltpu.make_async_copy(
                stage_v.at[b], out5d.at[bi, ti, pl.ds(hi, hh)],
                sem_w[b]).start()

        def drain_write(b):
            pltpu.make_async_copy(
                stage_v.at[b], out5d.at[0, 0, pl.ds(0, hh)],
                sem_w[b]).wait()

        ib = pl.multiple_of(idx0, _CHUNK)
        pltpu.sync_copy(idx_hbm.at[pl.ds(ib, per_w)], idx_all)

        def step(j, b, drain, fire):
            if drain:
                drain_write(b)
            if fire:
                fire_gather(j + 2, (b + 2) % _NBUF)
            wait_gather(b)
            repack(b)
            fire_write(j, b)

        fire_gather(0, 0)
        fire_gather(1, 1)
        for j in range(_NBUF):
            step(j, j, drain=False, fire=True)

        def body(k_, carry):
            for b in range(_NBUF):
                step(k_ * _NBUF + b, b, drain=True, fire=True)
            return carry

        lax.fori_loop(1, n_chunks // _NBUF - 1, body, 0)

        for j in range(n_chunks - _NBUF, n_chunks):
            step(j, j % _NBUF, drain=True, fire=(j + 2 < n_chunks))
        for b in range(_NBUF):
            drain_write(b)

    return k(tp, idx1d, dummy)


def kernel(actions, table):
    B, T, Hp, Wp = actions.shape
    n = B * T * Hp * Wp
    v, d = table.shape
    idx1d = actions.reshape(n).astype(jnp.int32)
    tp = jnp.concatenate(
        [jnp.repeat(table, v, axis=0), jnp.tile(table, (v, 1))], axis=1)
    dummy = jnp.zeros((_CHUNK // 2, 2 * d), jnp.float32)
    return _emb_lookup(tp, idx1d, dummy, n=n, d=d,
                       out_shape=(B, T, Hp, Wp, d))


# speedup vs baseline: 1.0127x; 1.0127x over previous
import functools

import jax
import jax.numpy as jnp
from jax import lax
from jax.experimental import pallas as pl
from jax.experimental.pallas import tpu as pltpu
from jax.experimental.pallas import tpu_sc as plsc

_INFO = plsc.get_sparse_core_info()
_NC, _NS = _INFO.num_cores, _INFO.num_subcores
_NW = _NC * _NS
_L = 16

_CHUNK = 256
_PAIRS = _CHUNK // 2
_NBUF = 4
_DO_GATHER = True
_DO_WRITE = True


@functools.partial(jax.jit, static_argnames=("n", "d"))
def _emb_lookup(tp, idx1d, dummy, *, n, d):
    per_w = n // _NW
    n_chunks = per_w // _CHUNK
    assert (n_chunks - 2 * _NBUF) % _NBUF == 0 and n_chunks > 3 * _NBUF
    mesh = plsc.VectorSubcoreMesh(core_axis_name="c", subcore_axis_name="s")

    @functools.partial(
        pl.kernel,
        mesh=mesh,
        out_type=jax.ShapeDtypeStruct((n // 2, 2 * d), jnp.float32),
        scratch_types=[
            pltpu.VMEM((per_w,), jnp.int32),
            pltpu.VMEM((_NBUF, _PAIRS), jnp.int32),
            pltpu.VMEM((_NBUF, _PAIRS, 2 * d), jnp.float32),
            pltpu.SemaphoreType.DMA,
            pltpu.SemaphoreType.DMA,
            pltpu.SemaphoreType.DMA,
            pltpu.SemaphoreType.DMA,
            pltpu.SemaphoreType.DMA,
            pltpu.SemaphoreType.DMA,
            pltpu.SemaphoreType.DMA,
            pltpu.SemaphoreType.DMA,
        ],
    )
    def k(tp_hbm, idx_hbm, dummy_hbm, out_hbm, idx_all, pid_v, pairs_f,
          *sems):
        sem_g = sems[:_NBUF]
        sem_w = sems[_NBUF:]
        wid = lax.axis_index("s") * _NC + lax.axis_index("c")
        idx0 = wid * per_w
        pair0 = idx0 // 2
        lane = lax.iota(jnp.int32, _L)
        low_half = lane < 8
        ev_sel = jnp.arange(0, 2 * _L, 2, dtype=jnp.int32) % _L
        od_sel = ev_sel + 1

        def deinterleave(vv, sel):
            return vv.at[sel].get(mode="promise_in_bounds")

        def fire_gather(j, b):
            base = j * _CHUNK
            for g in range(_PAIRS // _L):
                v0 = idx_all[pl.ds(base + 2 * _L * g, _L)]
                v1 = idx_all[pl.ds(base + 2 * _L * g + _L, _L)]
                ev = jnp.where(low_half, deinterleave(v0, ev_sel),
                               deinterleave(v1, ev_sel))
                od = jnp.where(low_half, deinterleave(v0, od_sel),
                               deinterleave(v1, od_sel))
                pid_v[b, pl.ds(g * _L, _L)] = ev * 8 + od
            if _DO_GATHER:
                pltpu.make_async_copy(
                    tp_hbm.at[pid_v.at[b]], pairs_f.at[b], sem_g[b]).start()

        def wait_gather(b):
            if _DO_GATHER:
                pltpu.make_async_copy(dummy_hbm, pairs_f.at[b],
                                      sem_g[b]).wait()

        def fire_write(j, b):
            pb = pl.multiple_of(pair0 + j * _PAIRS, _PAIRS)
            if _DO_WRITE:
                pltpu.make_async_copy(
                    pairs_f.at[b], out_hbm.at[pl.ds(pb, _PAIRS)],
                    sem_w[b]).start()

        def drain_write(b):
            if _DO_WRITE:
                pltpu.make_async_copy(
                    pairs_f.at[b], out_hbm.at[pl.ds(0, _PAIRS)],
                    sem_w[b]).wait()

        ib = pl.multiple_of(idx0, _CHUNK)
        pltpu.sync_copy(idx_hbm.at[pl.ds(ib, per_w)], idx_all)

        def step(j, b, drain, fire):
            if drain:
                drain_write(b)
            if fire:
                fire_gather(j + 2, (b + 2) % _NBUF)
            wait_gather(b)
            fire_write(j, b)

        fire_gather(0, 0)
        fire_gather(1, 1)
        for j in range(_NBUF):
            step(j, j, drain=False, fire=True)

        def body(k_, carry):
            for b in range(_NBUF):
                step(k_ * _NBUF + b, b, drain=True, fire=True)
            return carry

        lax.fori_loop(1, n_chunks // _NBUF - 1, body, 0)

        for j in range(n_chunks - _NBUF, n_chunks):
            step(j, j % _NBUF, drain=True, fire=(j + 2 < n_chunks))
        for b in range(_NBUF):
            drain_write(b)

    return k(tp, idx1d, dummy)


def kernel(actions, table):
    B, T, Hp, Wp = actions.shape
    n = B * T * Hp * Wp
    v, d = table.shape
    idx1d = actions.reshape(n).astype(jnp.int32)
    tp = jnp.concatenate(
        [jnp.repeat(table, v, axis=0), jnp.tile(table, (v, 1))], axis=1)
    dummy = jnp.zeros((_PAIRS, 2 * d), jnp.float32)
    out2 = _emb_lookup(tp, idx1d, dummy, n=n, d=d)
    return out2.reshape(B, T, Hp, Wp, d)
